# baseline (speedup 1.0000x reference)
import functools

import jax
import jax.numpy as jnp
from jax.experimental import pallas as pl
from jax.experimental.pallas import tpu as pltpu

NUM_EMBEDDINGS = 1000
EMBEDDING_DIM = 128
PADDING_IDX = 0

_BLOCK_M = 512


def _matmul_kernel(x_ref, w_ref, o_ref):
    w = w_ref[...]
    row_ids = jax.lax.broadcasted_iota(jnp.int32, w.shape, 0)
    w = jnp.where(row_ids == PADDING_IDX, 0.0, w).astype(jnp.bfloat16)
    x = x_ref[...].astype(jnp.bfloat16)
    o_ref[...] = jnp.dot(x, w, preferred_element_type=jnp.float32)


@functools.partial(jax.jit, static_argnames=())
def kernel(input, weight):
    b, l, v = input.shape
    d = weight.shape[1]
    m = b * l
    x = input.reshape(m, v)
    grid = (m // _BLOCK_M,)
    out = pl.pallas_call(
        _matmul_kernel,
        grid=grid,
        in_specs=[
            pl.BlockSpec((_BLOCK_M, v), lambda i: (i, 0)),
            pl.BlockSpec((v, d), lambda i: (0, 0)),
        ],
        out_specs=pl.BlockSpec((_BLOCK_M, d), lambda i: (i, 0)),
        out_shape=jax.ShapeDtypeStruct((m, d), jnp.float32),
        compiler_params=pltpu.CompilerParams(
            dimension_semantics=("arbitrary",),
        ),
    )(x, weight)
    return out.reshape(b, l, d)


# speedup vs baseline: 1.0966x; 1.0966x over previous
import functools

import jax
import jax.numpy as jnp
from jax.experimental import pallas as pl
from jax.experimental.pallas import tpu as pltpu

NUM_EMBEDDINGS = 1000
EMBEDDING_DIM = 128
PADDING_IDX = 0

_BLOCK_M = 2048


def _matmul_kernel(x_ref, w_ref, o_ref):
    w = w_ref[...]
    row_ids = jax.lax.broadcasted_iota(jnp.int32, w.shape, 0)
    w = jnp.where(row_ids == PADDING_IDX, 0.0, w).astype(jnp.bfloat16)
    x = x_ref[...].astype(jnp.bfloat16)
    o_ref[...] = jnp.dot(x, w, preferred_element_type=jnp.float32)


@functools.partial(jax.jit, static_argnames=())
def kernel(input, weight):
    b, l, v = input.shape
    d = weight.shape[1]
    m = b * l
    x = input.reshape(m, v)
    grid = (m // _BLOCK_M,)
    out = pl.pallas_call(
        _matmul_kernel,
        grid=grid,
        in_specs=[
            pl.BlockSpec((_BLOCK_M, v), lambda i: (i, 0)),
            pl.BlockSpec((v, d), lambda i: (0, 0)),
        ],
        out_specs=pl.BlockSpec((_BLOCK_M, d), lambda i: (i, 0)),
        out_shape=jax.ShapeDtypeStruct((m, d), jnp.float32),
        compiler_params=pltpu.CompilerParams(
            dimension_semantics=("arbitrary",),
        ),
    )(x, weight)
    return out.reshape(b, l, d)


# speedup vs baseline: 1.4585x; 1.3301x over previous
import jax
import jax.numpy as jnp
from jax.experimental import pallas as pl
from jax.experimental.pallas import tpu as pltpu

PADDING_IDX = 0

_BLOCK_B = 16


def _matmul_kernel(x_ref, w_ref, o_ref):
    w = w_ref[...]
    row_ids = jax.lax.broadcasted_iota(jnp.int32, w.shape, 0)
    w = jnp.where(row_ids == PADDING_IDX, 0.0, w).astype(jnp.bfloat16)
    for j in range(x_ref.shape[0]):
        x = x_ref[j].astype(jnp.bfloat16)
        o_ref[j] = jnp.dot(x, w, preferred_element_type=jnp.float32)


def kernel(input, weight):
    b, l, v = input.shape
    d = weight.shape[1]
    grid = (b // _BLOCK_B,)
    return pl.pallas_call(
        _matmul_kernel,
        grid=grid,
        in_specs=[
            pl.BlockSpec((_BLOCK_B, l, v), lambda i: (i, 0, 0)),
            pl.BlockSpec((v, d), lambda i: (0, 0)),
        ],
        out_specs=pl.BlockSpec((_BLOCK_B, l, d), lambda i: (i, 0, 0)),
        out_shape=jax.ShapeDtypeStruct((b, l, d), jnp.float32),
        compiler_params=pltpu.CompilerParams(
            dimension_semantics=("arbitrary",),
        ),
    )(input, weight)


# speedup vs baseline: 1.5414x; 1.0568x over previous
import jax
import jax.numpy as jnp
from jax.experimental import pallas as pl
from jax.experimental.pallas import tpu as pltpu

PADDING_IDX = 0

_BLOCK_B = 32


def _matmul_kernel(x_ref, w_ref, o_ref):
    w = w_ref[...]
    row_ids = jax.lax.broadcasted_iota(jnp.int32, w.shape, 0)
    w = jnp.where(row_ids == PADDING_IDX, 0.0, w).astype(jnp.bfloat16)
    for j in range(x_ref.shape[0]):
        x = x_ref[j].astype(jnp.bfloat16)
        o_ref[j] = jnp.dot(x, w, preferred_element_type=jnp.float32)


def kernel(input, weight):
    b, l, v = input.shape
    d = weight.shape[1]
    grid = (b // _BLOCK_B,)
    return pl.pallas_call(
        _matmul_kernel,
        grid=grid,
        in_specs=[
            pl.BlockSpec((_BLOCK_B, l, v), lambda i: (i, 0, 0)),
            pl.BlockSpec((v, d), lambda i: (0, 0)),
        ],
        out_specs=pl.BlockSpec((_BLOCK_B, l, d), lambda i: (i, 0, 0)),
        out_shape=jax.ShapeDtypeStruct((b, l, d), jnp.float32),
        compiler_params=pltpu.CompilerParams(
            dimension_semantics=("parallel",),
        ),
    )(input, weight)


# speedup vs baseline: 1.5858x; 1.0288x over previous
import jax
import jax.numpy as jnp
from jax.experimental import pallas as pl
from jax.experimental.pallas import tpu as pltpu

PADDING_IDX = 0

_BLOCK_B = 64


def _matmul_kernel(x_ref, w_ref, o_ref):
    w = w_ref[...]
    row_ids = jax.lax.broadcasted_iota(jnp.int32, w.shape, 0)
    w = jnp.where(row_ids == PADDING_IDX, 0.0, w).astype(jnp.bfloat16)
    for j in range(x_ref.shape[0]):
        x = x_ref[j].astype(jnp.bfloat16)
        o_ref[j] = jnp.dot(x, w, preferred_element_type=jnp.float32)


def kernel(input, weight):
    b, l, v = input.shape
    d = weight.shape[1]
    grid = (b // _BLOCK_B,)
    return pl.pallas_call(
        _matmul_kernel,
        grid=grid,
        in_specs=[
            pl.BlockSpec((_BLOCK_B, l, v), lambda i: (i, 0, 0)),
            pl.BlockSpec((v, d), lambda i: (0, 0)),
        ],
        out_specs=pl.BlockSpec((_BLOCK_B, l, d), lambda i: (i, 0, 0)),
        out_shape=jax.ShapeDtypeStruct((b, l, d), jnp.float32),
        compiler_params=pltpu.CompilerParams(
            dimension_semantics=("parallel",),
        ),
    )(input, weight)
